# speedup vs baseline: 4.7216x; 4.7216x over previous
import functools

import jax
import jax.numpy as jnp
from jax import lax
from jax.experimental import pallas as pl
from jax.experimental.pallas import tpu as pltpu
from jax.experimental.pallas import tpu_sc as plsc

N = 10000
E = 320000
F = 128
L = 2048
NPAD = 10240
NC = 2
NS = 16
NW = NC * NS
EDGES_PER_W = E // NW
CHUNK = 80
N_CHUNKS = EDGES_PER_W // CHUNK
ROWS_PER_S = NPAD // NS
ZCH = 64

_mesh = plsc.VectorSubcoreMesh(core_axis_name="c", subcore_axis_name="s")


def _sc_embed(table, nf):
    @functools.partial(
        pl.kernel,
        out_type=jax.ShapeDtypeStruct((NPAD, F), jnp.float32),
        mesh=_mesh,
        scratch_types=[
            pltpu.VMEM((CHUNK,), jnp.int32),
            pltpu.VMEM((CHUNK, F), jnp.float32),
            pltpu.SemaphoreType.DMA,
        ],
    )
    def k(table_hbm, nf_hbm, out_hbm, idx_v, rows_v, sem):
        w = lax.axis_index("c") * NS + lax.axis_index("s")
        rows_per_w = NPAD // NW

        @pl.loop(0, rows_per_w // CHUNK)
        def _(j):
            base = w * rows_per_w + j * CHUNK
            pltpu.sync_copy(nf_hbm.at[pl.ds(base, CHUNK)], idx_v)
            pltpu.async_copy(table_hbm.at[idx_v], rows_v, sem).wait()
            pltpu.sync_copy(rows_v, out_hbm.at[pl.ds(base, CHUNK)])

    return k(table, nf)


def _sc_scatter(emb, src, dst):
    @functools.partial(
        pl.kernel,
        out_type=jax.ShapeDtypeStruct((NC, NPAD, F), jnp.float32),
        mesh=_mesh,
        scratch_types=[
            pltpu.VMEM((CHUNK,), jnp.int32),
            pltpu.VMEM((CHUNK,), jnp.int32),
            pltpu.VMEM((CHUNK, F), jnp.float32),
            pltpu.VMEM((ZCH, F), jnp.float32),
            pltpu.VMEM_SHARED((NPAD, F), jnp.float32),
            pltpu.SemaphoreType.DMA,
        ],
    )
    def k(emb_hbm, src_hbm, dst_hbm, out_hbm, src_v, dst_v, rows_v, zero_v,
          acc_sh, sem):
        c = lax.axis_index("c")
        s = lax.axis_index("s")

        @pl.loop(0, ZCH)
        def _(r):
            @pl.loop(0, F // 16)
            def _(j):
                zero_v[r, pl.ds(j * 16, 16)] = jnp.zeros((16,), jnp.float32)

        @pl.loop(0, ROWS_PER_S // ZCH)
        def _(t):
            pltpu.sync_copy(
                zero_v, acc_sh.at[pl.ds(s * ROWS_PER_S + t * ZCH, ZCH)])

        plsc.subcore_barrier()

        w = c * NS + s

        @pl.loop(0, N_CHUNKS)
        def _(i):
            base = w * EDGES_PER_W + i * CHUNK
            pltpu.sync_copy(src_hbm.at[pl.ds(base, CHUNK)], src_v)
            pltpu.sync_copy(dst_hbm.at[pl.ds(base, CHUNK)], dst_v)
            pltpu.async_copy(emb_hbm.at[src_v], rows_v, sem).wait()
            pltpu.sync_copy(rows_v, acc_sh.at[dst_v], add=True)

        plsc.subcore_barrier()

        pltpu.sync_copy(
            acc_sh.at[pl.ds(s * ROWS_PER_S, ROWS_PER_S)],
            out_hbm.at[c].at[pl.ds(s * ROWS_PER_S, ROWS_PER_S)])

    return k(emb, src, dst)


_BR = 256
_NB = NPAD // _BR


def _tc_hidden(p, emb, w_h, b_h):
    def body(p_ref, emb_ref, w_ref, b_ref, h_ref):
        agg = emb_ref[...] + p_ref[0] + p_ref[1]
        h = jnp.dot(agg, w_ref[...], preferred_element_type=jnp.float32)
        h_ref[...] = jnp.maximum(h + b_ref[...], 0.0)

    return pl.pallas_call(
        body,
        grid=(_NB,),
        in_specs=[
            pl.BlockSpec((NC, _BR, F), lambda i: (0, i, 0)),
            pl.BlockSpec((_BR, F), lambda i: (i, 0)),
            pl.BlockSpec((F, F), lambda i: (0, 0)),
            pl.BlockSpec((1, F), lambda i: (0, 0)),
        ],
        out_specs=pl.BlockSpec((_BR, F), lambda i: (i, 0)),
        out_shape=jax.ShapeDtypeStruct((NPAD, F), jnp.float32),
    )(p, emb, w_h, b_h)


def _tc_fingerprint(h, w_o, b_o, f_in, final):
    def body(h_ref, w_ref, b_ref, fin_ref, fout_ref, acc_ref):
        i = pl.program_id(0)

        @pl.when(i == 0)
        def _():
            acc_ref[...] = fin_ref[...]

        z = jnp.dot(h_ref[...], w_ref[...], preferred_element_type=jnp.float32)
        z = z + b_ref[...]
        m = jnp.max(z, axis=-1, keepdims=True)
        e = jnp.exp(z - m)
        sm = e / jnp.sum(e, axis=-1, keepdims=True)
        row = i * _BR + lax.broadcasted_iota(jnp.int32, (_BR, 1), 0)
        sm = jnp.where(row < N, sm, 0.0)
        acc_ref[...] += jnp.sum(sm, axis=0, keepdims=True)

        @pl.when(i == _NB - 1)
        def _():
            if final:
                t = acc_ref[...]
                tm = jnp.max(t, axis=-1, keepdims=True)
                te = jnp.exp(t - tm)
                fout_ref[...] = te / jnp.sum(te, axis=-1, keepdims=True)
            else:
                fout_ref[...] = acc_ref[...]

    return pl.pallas_call(
        body,
        grid=(_NB,),
        in_specs=[
            pl.BlockSpec((_BR, F), lambda i: (i, 0)),
            pl.BlockSpec((F, L), lambda i: (0, 0)),
            pl.BlockSpec((1, L), lambda i: (0, 0)),
            pl.BlockSpec((1, L), lambda i: (0, 0)),
        ],
        out_specs=pl.BlockSpec((1, L), lambda i: (0, 0)),
        out_shape=jax.ShapeDtypeStruct((1, L), jnp.float32),
        scratch_shapes=[pltpu.VMEM((1, L), jnp.float32)],
    )(h, w_o, b_o, f_in)


def kernel(node_feature, edge_index, embedding_table, W_h, b_h, W_o, b_o):
    nf = jnp.pad(node_feature.astype(jnp.int32), (0, NPAD - N))
    src = edge_index[0].astype(jnp.int32)
    dst = edge_index[1].astype(jnp.int32)

    emb = _sc_embed(embedding_table, nf)
    f = jnp.zeros((1, L), jnp.float32)
    n_rounds = W_h.shape[0]
    for l in range(n_rounds):
        p = _sc_scatter(emb, src, dst)
        h = _tc_hidden(p, emb, W_h[l], b_h[l].reshape(1, F))
        f = _tc_fingerprint(h, W_o[l], b_o[l].reshape(1, L), f,
                            final=(l == n_rounds - 1))
        emb = h
    return f.reshape(L)


# speedup vs baseline: 9.6280x; 2.0391x over previous
import functools

import jax
import jax.numpy as jnp
from jax import lax
from jax.experimental import pallas as pl
from jax.experimental.pallas import tpu as pltpu
from jax.experimental.pallas import tpu_sc as plsc

N = 10000
E = 320000
F = 128
L = 2048
NPAD = 10240
NC = 2
NS = 16
NW = NC * NS
EDGES_PER_W = E // NW
CHUNK = 80
N_CHUNKS = EDGES_PER_W // CHUNK
ROWS_PER_S = NPAD // NS

_mesh = plsc.VectorSubcoreMesh(core_axis_name="c", subcore_axis_name="s")


def _sc_embed(table, nf):
    @functools.partial(
        pl.kernel,
        out_type=jax.ShapeDtypeStruct((NPAD, F), jnp.float32),
        mesh=_mesh,
        scratch_types=[
            pltpu.VMEM((CHUNK,), jnp.int32),
            pltpu.VMEM((CHUNK, F), jnp.float32),
            pltpu.SemaphoreType.DMA,
        ],
    )
    def k(table_hbm, nf_hbm, out_hbm, idx_v, rows_v, sem):
        w = lax.axis_index("c") * NS + lax.axis_index("s")
        rows_per_w = NPAD // NW

        @pl.loop(0, rows_per_w // CHUNK)
        def _(j):
            base = w * rows_per_w + j * CHUNK
            pltpu.sync_copy(nf_hbm.at[pl.ds(base, CHUNK)], idx_v)
            pltpu.async_copy(table_hbm.at[idx_v], rows_v, sem).wait()
            pltpu.sync_copy(rows_v, out_hbm.at[pl.ds(base, CHUNK)])

    return k(table, nf)


def _sc_scatter(emb, src2, dst3, zrows):
    @functools.partial(
        pl.kernel,
        out_type=jax.ShapeDtypeStruct((NC, NPAD, F), jnp.float32),
        mesh=_mesh,
        scratch_types=[
            pltpu.VMEM((EDGES_PER_W,), jnp.int32),
            pltpu.VMEM((N_CHUNKS, CHUNK), jnp.int32),
            pltpu.VMEM((CHUNK, F), jnp.float32),
            pltpu.VMEM((CHUNK, F), jnp.float32),
            pltpu.VMEM_SHARED((NPAD, F), jnp.float32),
            pltpu.SemaphoreType.DMA,
            pltpu.SemaphoreType.DMA,
            pltpu.SemaphoreType.DMA,
        ],
    )
    def k(emb_hbm, src_hbm, dst_hbm, z_hbm, out_hbm, src_v, dst_v, rows_a,
          rows_b, acc_sh, sem_i, sem_a, sem_b):
        c = lax.axis_index("c")
        s = lax.axis_index("s")
        w = c * NS + s

        cp_src = pltpu.make_async_copy(src_hbm.at[w], src_v, sem_i)
        cp_dst = pltpu.make_async_copy(dst_hbm.at[w], dst_v, sem_i)
        cp_z = pltpu.make_async_copy(
            z_hbm, acc_sh.at[pl.ds(s * ROWS_PER_S, ROWS_PER_S)], sem_i)
        cp_src.start()
        cp_dst.start()
        cp_z.start()
        cp_src.wait()
        cp_dst.wait()
        cp_z.wait()
        plsc.subcore_barrier()

        def gather(j, buf, sem):
            return pltpu.make_async_copy(
                emb_hbm.at[src_v.at[pl.ds(j * CHUNK, CHUNK)]], buf, sem)

        def scatter_add(j, buf):
            pltpu.sync_copy(buf, acc_sh.at[dst_v.at[j]], add=True)

        gather(0, rows_a, sem_a).start()

        @pl.loop(0, N_CHUNKS - 1, step=2)
        def _(i):
            gather(i + 1, rows_b, sem_b).start()
            gather(i, rows_a, sem_a).wait()
            scatter_add(i, rows_a)
            @pl.when(i + 2 < N_CHUNKS)
            def _():
                gather(i + 2, rows_a, sem_a).start()
            gather(i + 1, rows_b, sem_b).wait()
            scatter_add(i + 1, rows_b)

        gather(N_CHUNKS - 1, rows_a, sem_a).wait()
        scatter_add(N_CHUNKS - 1, rows_a)

        plsc.subcore_barrier()

        pltpu.sync_copy(
            acc_sh.at[pl.ds(s * ROWS_PER_S, ROWS_PER_S)],
            out_hbm.at[c].at[pl.ds(s * ROWS_PER_S, ROWS_PER_S)])

    return k(emb, src2, dst3, zrows)


_BR = 256
_NB = NPAD // _BR


def _tc_hidden(p, emb, w_h, b_h):
    def body(p_ref, emb_ref, w_ref, b_ref, h_ref):
        agg = emb_ref[...] + p_ref[0] + p_ref[1]
        h = jnp.dot(agg, w_ref[...], preferred_element_type=jnp.float32)
        h_ref[...] = jnp.maximum(h + b_ref[...], 0.0)

    return pl.pallas_call(
        body,
        grid=(_NB,),
        in_specs=[
            pl.BlockSpec((NC, _BR, F), lambda i: (0, i, 0)),
            pl.BlockSpec((_BR, F), lambda i: (i, 0)),
            pl.BlockSpec((F, F), lambda i: (0, 0)),
            pl.BlockSpec((1, F), lambda i: (0, 0)),
        ],
        out_specs=pl.BlockSpec((_BR, F), lambda i: (i, 0)),
        out_shape=jax.ShapeDtypeStruct((NPAD, F), jnp.float32),
    )(p, emb, w_h, b_h)


def _tc_fingerprint(h, w_o, b_o, f_in, final):
    def body(h_ref, w_ref, b_ref, fin_ref, fout_ref, acc_ref):
        i = pl.program_id(0)

        @pl.when(i == 0)
        def _():
            acc_ref[...] = fin_ref[...]

        z = jnp.dot(h_ref[...], w_ref[...], preferred_element_type=jnp.float32)
        z = z + b_ref[...]
        m = jnp.max(z, axis=-1, keepdims=True)
        e = jnp.exp(z - m)
        sm = e / jnp.sum(e, axis=-1, keepdims=True)
        row = i * _BR + lax.broadcasted_iota(jnp.int32, (_BR, 1), 0)
        sm = jnp.where(row < N, sm, 0.0)
        acc_ref[...] += jnp.sum(sm, axis=0, keepdims=True)

        @pl.when(i == _NB - 1)
        def _():
            if final:
                t = acc_ref[...]
                tm = jnp.max(t, axis=-1, keepdims=True)
                te = jnp.exp(t - tm)
                fout_ref[...] = te / jnp.sum(te, axis=-1, keepdims=True)
            else:
                fout_ref[...] = acc_ref[...]

    return pl.pallas_call(
        body,
        grid=(_NB,),
        in_specs=[
            pl.BlockSpec((_BR, F), lambda i: (i, 0)),
            pl.BlockSpec((F, L), lambda i: (0, 0)),
            pl.BlockSpec((1, L), lambda i: (0, 0)),
            pl.BlockSpec((1, L), lambda i: (0, 0)),
        ],
        out_specs=pl.BlockSpec((1, L), lambda i: (0, 0)),
        out_shape=jax.ShapeDtypeStruct((1, L), jnp.float32),
        scratch_shapes=[pltpu.VMEM((1, L), jnp.float32)],
    )(h, w_o, b_o, f_in)


def kernel(node_feature, edge_index, embedding_table, W_h, b_h, W_o, b_o):
    nf = jnp.pad(node_feature.astype(jnp.int32), (0, NPAD - N))
    src = edge_index[0].astype(jnp.int32).reshape(NW, EDGES_PER_W)
    dst = edge_index[1].astype(jnp.int32).reshape(NW, N_CHUNKS, CHUNK)
    zrows = jnp.zeros((ROWS_PER_S, F), jnp.float32)

    emb = _sc_embed(embedding_table, nf)
    f = jnp.zeros((1, L), jnp.float32)
    n_rounds = W_h.shape[0]
    for l in range(n_rounds):
        p = _sc_scatter(emb, src, dst, zrows)
        h = _tc_hidden(p, emb, W_h[l], b_h[l].reshape(1, F))
        f = _tc_fingerprint(h, W_o[l], b_o[l].reshape(1, L), f,
                            final=(l == n_rounds - 1))
        emb = h
    return f.reshape(L)


# speedup vs baseline: 10.6118x; 1.1022x over previous
import functools

import jax
import jax.numpy as jnp
from jax import lax
from jax.experimental import pallas as pl
from jax.experimental.pallas import tpu as pltpu
from jax.experimental.pallas import tpu_sc as plsc

N = 10000
E = 320000
F = 128
L = 2048
NPAD = 10240
NC = 2
NS = 16
NW = NC * NS
EDGES_PER_W = E // NW
CHUNK = 80
N_CHUNKS = EDGES_PER_W // CHUNK
ROWS_PER_S = NPAD // NS

_mesh = plsc.VectorSubcoreMesh(core_axis_name="c", subcore_axis_name="s")


def _sc_embed(table, nf):
    @functools.partial(
        pl.kernel,
        out_type=jax.ShapeDtypeStruct((NPAD, F), jnp.float32),
        mesh=_mesh,
        scratch_types=[
            pltpu.VMEM((CHUNK,), jnp.int32),
            pltpu.VMEM((CHUNK, F), jnp.float32),
            pltpu.SemaphoreType.DMA,
        ],
    )
    def k(table_hbm, nf_hbm, out_hbm, idx_v, rows_v, sem):
        w = lax.axis_index("c") * NS + lax.axis_index("s")
        rows_per_w = NPAD // NW

        @pl.loop(0, rows_per_w // CHUNK)
        def _(j):
            base = w * rows_per_w + j * CHUNK
            pltpu.sync_copy(nf_hbm.at[pl.ds(base, CHUNK)], idx_v)
            pltpu.async_copy(table_hbm.at[idx_v], rows_v, sem).wait()
            pltpu.sync_copy(rows_v, out_hbm.at[pl.ds(base, CHUNK)])

    return k(table, nf)


def _sc_scatter(emb, src2, dst3, zrows):
    @functools.partial(
        pl.kernel,
        out_type=jax.ShapeDtypeStruct((NC, NPAD, F), jnp.float32),
        mesh=_mesh,
        scratch_types=[
            pltpu.VMEM((EDGES_PER_W,), jnp.int32),
            pltpu.VMEM((N_CHUNKS, CHUNK), jnp.int32),
            pltpu.VMEM((CHUNK, F), jnp.float32),
            pltpu.VMEM((CHUNK, F), jnp.float32),
            pltpu.VMEM_SHARED((NPAD, F), jnp.float32),
            pltpu.SemaphoreType.DMA,
            pltpu.SemaphoreType.DMA,
            pltpu.SemaphoreType.DMA,
        ],
    )
    def k(emb_hbm, src_hbm, dst_hbm, z_hbm, out_hbm, src_v, dst_v, rows_a,
          rows_b, acc_sh, sem_i, sem_a, sem_b):
        c = lax.axis_index("c")
        s = lax.axis_index("s")
        w = c * NS + s

        cp_src = pltpu.make_async_copy(src_hbm.at[w], src_v, sem_i)
        cp_dst = pltpu.make_async_copy(dst_hbm.at[w], dst_v, sem_i)
        cp_z = pltpu.make_async_copy(
            z_hbm, acc_sh.at[pl.ds(s * ROWS_PER_S, ROWS_PER_S)], sem_i)
        cp_src.start()
        cp_dst.start()
        cp_z.start()
        cp_src.wait()
        cp_dst.wait()
        cp_z.wait()
        plsc.subcore_barrier()

        def gather(j, buf, sem):
            return pltpu.make_async_copy(
                emb_hbm.at[src_v.at[pl.ds(j * CHUNK, CHUNK)]], buf, sem)

        def scatter_add(j, buf):
            pltpu.sync_copy(buf, acc_sh.at[dst_v.at[j]], add=True)

        gather(0, rows_a, sem_a).start()

        @pl.loop(0, N_CHUNKS - 1, step=2)
        def _(i):
            gather(i + 1, rows_b, sem_b).start()
            gather(i, rows_a, sem_a).wait()
            scatter_add(i, rows_a)
            @pl.when(i + 2 < N_CHUNKS)
            def _():
                gather(i + 2, rows_a, sem_a).start()
            gather(i + 1, rows_b, sem_b).wait()
            scatter_add(i + 1, rows_b)

        gather(N_CHUNKS - 1, rows_a, sem_a).wait()
        scatter_add(N_CHUNKS - 1, rows_a)

        plsc.subcore_barrier()

        pltpu.sync_copy(
            acc_sh.at[pl.ds(s * ROWS_PER_S, ROWS_PER_S)],
            out_hbm.at[c].at[pl.ds(s * ROWS_PER_S, ROWS_PER_S)])

    return k(emb, src2, dst3, zrows)


_BRA = 1024
_NBA = NPAD // _BRA
_BRB = 512
_NBB = NPAD // _BRB


def _tc_hidden(p, emb, w_h, b_h):
    def body(p_ref, emb_ref, w_ref, b_ref, h_ref):
        agg = emb_ref[...] + p_ref[0] + p_ref[1]
        h = jnp.dot(agg, w_ref[...], preferred_element_type=jnp.float32)
        h_ref[...] = jnp.maximum(h + b_ref[...], 0.0)

    return pl.pallas_call(
        body,
        grid=(_NBA,),
        in_specs=[
            pl.BlockSpec((NC, _BRA, F), lambda i: (0, i, 0)),
            pl.BlockSpec((_BRA, F), lambda i: (i, 0)),
            pl.BlockSpec((F, F), lambda i: (0, 0)),
            pl.BlockSpec((1, F), lambda i: (0, 0)),
        ],
        out_specs=pl.BlockSpec((_BRA, F), lambda i: (i, 0)),
        out_shape=jax.ShapeDtypeStruct((NPAD, F), jnp.float32),
    )(p, emb, w_h, b_h)


def _tc_fingerprint(h, w_o, b_o, f_in, final):
    def body(h_ref, w_ref, b_ref, fin_ref, fout_ref, acc_ref):
        i = pl.program_id(0)

        @pl.when(i == 0)
        def _():
            acc_ref[...] = fin_ref[...]

        z = jnp.dot(h_ref[...], w_ref[...], preferred_element_type=jnp.float32)
        z = z + b_ref[...]
        m = jnp.max(z, axis=-1, keepdims=True)
        e = jnp.exp(z - m)
        sm = e * (1.0 / jnp.sum(e, axis=-1, keepdims=True))

        @pl.when(i < _NBB - 1)
        def _():
            acc_ref[...] += jnp.sum(sm, axis=0, keepdims=True)

        @pl.when(i == _NBB - 1)
        def _():
            row = i * _BRB + lax.broadcasted_iota(jnp.int32, (_BRB, 1), 0)
            smm = jnp.where(row < N, sm, 0.0)
            acc = acc_ref[...] + jnp.sum(smm, axis=0, keepdims=True)
            if final:
                tm = jnp.max(acc, axis=-1, keepdims=True)
                te = jnp.exp(acc - tm)
                fout_ref[...] = te * (1.0 / jnp.sum(te, axis=-1, keepdims=True))
            else:
                fout_ref[...] = acc

    return pl.pallas_call(
        body,
        grid=(_NBB,),
        in_specs=[
            pl.BlockSpec((_BRB, F), lambda i: (i, 0)),
            pl.BlockSpec((F, L), lambda i: (0, 0)),
            pl.BlockSpec((1, L), lambda i: (0, 0)),
            pl.BlockSpec((1, L), lambda i: (0, 0)),
        ],
        out_specs=pl.BlockSpec((1, L), lambda i: (0, 0)),
        out_shape=jax.ShapeDtypeStruct((1, L), jnp.float32),
        scratch_shapes=[pltpu.VMEM((1, L), jnp.float32)],
    )(h, w_o, b_o, f_in)


def kernel(node_feature, edge_index, embedding_table, W_h, b_h, W_o, b_o):
    nf = jnp.pad(node_feature.astype(jnp.int32), (0, NPAD - N))
    src = edge_index[0].astype(jnp.int32).reshape(NW, EDGES_PER_W)
    dst = edge_index[1].astype(jnp.int32).reshape(NW, N_CHUNKS, CHUNK)
    zrows = jnp.zeros((ROWS_PER_S, F), jnp.float32)

    emb = _sc_embed(embedding_table, nf)
    f = jnp.zeros((1, L), jnp.float32)
    n_rounds = W_h.shape[0]
    for l in range(n_rounds):
        p = _sc_scatter(emb, src, dst, zrows)
        h = _tc_hidden(p, emb, W_h[l], b_h[l].reshape(1, F))
        f = _tc_fingerprint(h, W_o[l], b_o[l].reshape(1, L), f,
                            final=(l == n_rounds - 1))
        emb = h
    return f.reshape(L)


# speedup vs baseline: 10.7927x; 1.0170x over previous
import functools

import jax
import jax.numpy as jnp
from jax import lax
from jax.experimental import pallas as pl
from jax.experimental.pallas import tpu as pltpu
from jax.experimental.pallas import tpu_sc as plsc

N = 10000
E = 320000
F = 128
L = 2048
NPAD = 10240
NC = 2
NS = 16
NW = NC * NS
EDGES_PER_W = E // NW
CHUNK = 80
N_CHUNKS = EDGES_PER_W // CHUNK
ROWS_PER_S = NPAD // NS

_mesh = plsc.VectorSubcoreMesh(core_axis_name="c", subcore_axis_name="s")


def _sc_embed(table, nf):
    @functools.partial(
        pl.kernel,
        out_type=jax.ShapeDtypeStruct((NPAD, F), jnp.float32),
        mesh=_mesh,
        scratch_types=[
            pltpu.VMEM((CHUNK,), jnp.int32),
            pltpu.VMEM((CHUNK, F), jnp.float32),
            pltpu.SemaphoreType.DMA,
        ],
    )
    def k(table_hbm, nf_hbm, out_hbm, idx_v, rows_v, sem):
        w = lax.axis_index("c") * NS + lax.axis_index("s")
        rows_per_w = NPAD // NW

        @pl.loop(0, rows_per_w // CHUNK)
        def _(j):
            base = w * rows_per_w + j * CHUNK
            pltpu.sync_copy(nf_hbm.at[pl.ds(base, CHUNK)], idx_v)
            pltpu.async_copy(table_hbm.at[idx_v], rows_v, sem).wait()
            pltpu.sync_copy(rows_v, out_hbm.at[pl.ds(base, CHUNK)])

    return k(table, nf)


def _sc_scatter(emb, src2, dst3, zrows):
    @functools.partial(
        pl.kernel,
        out_type=jax.ShapeDtypeStruct((NC, NPAD, F), jnp.float32),
        mesh=_mesh,
        scratch_types=[
            pltpu.VMEM((EDGES_PER_W,), jnp.int32),
            pltpu.VMEM((N_CHUNKS, CHUNK), jnp.int32),
            pltpu.VMEM((CHUNK, F), jnp.float32),
            pltpu.VMEM((CHUNK, F), jnp.float32),
            pltpu.VMEM_SHARED((NPAD, F), jnp.float32),
            pltpu.SemaphoreType.DMA,
            pltpu.SemaphoreType.DMA,
            pltpu.SemaphoreType.DMA,
        ],
    )
    def k(emb_hbm, src_hbm, dst_hbm, z_hbm, out_hbm, src_v, dst_v, rows_a,
          rows_b, acc_sh, sem_i, sem_a, sem_b):
        c = lax.axis_index("c")
        s = lax.axis_index("s")
        w = c * NS + s

        cp_src = pltpu.make_async_copy(src_hbm.at[w], src_v, sem_i)
        cp_dst = pltpu.make_async_copy(dst_hbm.at[w], dst_v, sem_i)
        cp_z = pltpu.make_async_copy(
            z_hbm, acc_sh.at[pl.ds(s * ROWS_PER_S, ROWS_PER_S)], sem_i)
        cp_src.start()
        cp_dst.start()
        cp_z.start()
        cp_src.wait()
        cp_dst.wait()
        cp_z.wait()
        plsc.subcore_barrier()

        def gather(j, buf, sem):
            return pltpu.make_async_copy(
                emb_hbm.at[src_v.at[pl.ds(j * CHUNK, CHUNK)]], buf, sem)

        def scatter_add(j, buf):
            pltpu.sync_copy(buf, acc_sh.at[dst_v.at[j]], add=True)

        gather(0, rows_a, sem_a).start()

        @pl.loop(0, N_CHUNKS - 1, step=2)
        def _(i):
            gather(i + 1, rows_b, sem_b).start()
            gather(i, rows_a, sem_a).wait()
            scatter_add(i, rows_a)
            @pl.when(i + 2 < N_CHUNKS)
            def _():
                gather(i + 2, rows_a, sem_a).start()
            gather(i + 1, rows_b, sem_b).wait()
            scatter_add(i + 1, rows_b)

        gather(N_CHUNKS - 1, rows_a, sem_a).wait()
        scatter_add(N_CHUNKS - 1, rows_a)

        plsc.subcore_barrier()

        pltpu.sync_copy(
            acc_sh.at[pl.ds(s * ROWS_PER_S, ROWS_PER_S)],
            out_hbm.at[c].at[pl.ds(s * ROWS_PER_S, ROWS_PER_S)])

    return k(emb, src2, dst3, zrows)


_BRA = 1024
_NBA = NPAD // _BRA
_BRB = 512
_NBB = NPAD // _BRB


def _tc_embed(nf2, table):
    def body(nf_ref, t_ref, o_ref):
        oh = (nf_ref[...] == lax.broadcasted_iota(jnp.int32, (_BRA, F), 1))
        o_ref[...] = jnp.dot(oh.astype(jnp.float32), t_ref[...],
                             preferred_element_type=jnp.float32)

    return pl.pallas_call(
        body,
        grid=(_NBA,),
        in_specs=[
            pl.BlockSpec((_BRA, 1), lambda i: (i, 0)),
            pl.BlockSpec((F, F), lambda i: (0, 0)),
        ],
        out_specs=pl.BlockSpec((_BRA, F), lambda i: (i, 0)),
        out_shape=jax.ShapeDtypeStruct((NPAD, F), jnp.float32),
    )(nf2, table)


def _tc_hidden(p, emb, w_h, b_h):
    def body(p_ref, emb_ref, w_ref, b_ref, h_ref):
        agg = emb_ref[...] + p_ref[0] + p_ref[1]
        h = jnp.dot(agg, w_ref[...], preferred_element_type=jnp.float32)
        h_ref[...] = jnp.maximum(h + b_ref[...], 0.0)

    return pl.pallas_call(
        body,
        grid=(_NBA,),
        in_specs=[
            pl.BlockSpec((NC, _BRA, F), lambda i: (0, i, 0)),
            pl.BlockSpec((_BRA, F), lambda i: (i, 0)),
            pl.BlockSpec((F, F), lambda i: (0, 0)),
            pl.BlockSpec((1, F), lambda i: (0, 0)),
        ],
        out_specs=pl.BlockSpec((_BRA, F), lambda i: (i, 0)),
        out_shape=jax.ShapeDtypeStruct((NPAD, F), jnp.float32),
    )(p, emb, w_h, b_h)


def _tc_fingerprint(h, w_o, b_o, f_in, final):
    def body(h_ref, w_ref, b_ref, fin_ref, fout_ref, acc_ref):
        i = pl.program_id(0)

        @pl.when(i == 0)
        def _():
            acc_ref[...] = fin_ref[...]

        z = jnp.dot(h_ref[...], w_ref[...], preferred_element_type=jnp.float32)
        z = z + b_ref[...]
        m = jnp.max(z, axis=-1, keepdims=True)
        e = jnp.exp(z - m)
        sm = e * (1.0 / jnp.sum(e, axis=-1, keepdims=True))

        @pl.when(i < _NBB - 1)
        def _():
            acc_ref[...] += jnp.sum(sm, axis=0, keepdims=True)

        @pl.when(i == _NBB - 1)
        def _():
            row = i * _BRB + lax.broadcasted_iota(jnp.int32, (_BRB, 1), 0)
            smm = jnp.where(row < N, sm, 0.0)
            acc = acc_ref[...] + jnp.sum(smm, axis=0, keepdims=True)
            if final:
                tm = jnp.max(acc, axis=-1, keepdims=True)
                te = jnp.exp(acc - tm)
                fout_ref[...] = te * (1.0 / jnp.sum(te, axis=-1, keepdims=True))
            else:
                fout_ref[...] = acc

    return pl.pallas_call(
        body,
        grid=(_NBB,),
        in_specs=[
            pl.BlockSpec((_BRB, F), lambda i: (i, 0)),
            pl.BlockSpec((F, L), lambda i: (0, 0)),
            pl.BlockSpec((1, L), lambda i: (0, 0)),
            pl.BlockSpec((1, L), lambda i: (0, 0)),
        ],
        out_specs=pl.BlockSpec((1, L), lambda i: (0, 0)),
        out_shape=jax.ShapeDtypeStruct((1, L), jnp.float32),
        scratch_shapes=[pltpu.VMEM((1, L), jnp.float32)],
    )(h, w_o, b_o, f_in)


def kernel(node_feature, edge_index, embedding_table, W_h, b_h, W_o, b_o):
    nf = jnp.pad(node_feature.astype(jnp.int32), (0, NPAD - N))
    src = edge_index[0].astype(jnp.int32).reshape(NW, EDGES_PER_W)
    dst = edge_index[1].astype(jnp.int32).reshape(NW, N_CHUNKS, CHUNK)
    zrows = jnp.zeros((ROWS_PER_S, F), jnp.float32)

    emb = _tc_embed(nf.reshape(NPAD, 1), embedding_table)
    f = jnp.zeros((1, L), jnp.float32)
    n_rounds = W_h.shape[0]
    for l in range(n_rounds):
        p = _sc_scatter(emb, src, dst, zrows)
        h = _tc_hidden(p, emb, W_h[l], b_h[l].reshape(1, F))
        f = _tc_fingerprint(h, W_o[l], b_o[l].reshape(1, L), f,
                            final=(l == n_rounds - 1))
        emb = h
    return f.reshape(L)
